# baseline (speedup 1.0000x reference)
import functools
import jax
import jax.numpy as jnp
from jax import lax
from jax.experimental import pallas as pl
from jax.experimental.pallas import tpu as pltpu
from jax.experimental.pallas import tpu_sc as plsc

N = 10000
E = 320000
H = 64
G = 128
NC = 2
NS = 16
NW = NC * NS
CH = 128
NCHUNK = -(-E // (NW * CH))
EPAD = NW * CH * NCHUNK - E
NA = N + 16
RPW = 624
RTAIL = NA - NS * RPW

def _fill_rows(ref, n_rows, n_cols, value):
    v = jnp.full((16,), value, dtype=ref.dtype)

    def body(i, _):
        for j in range(n_cols // 16):
            ref[i, pl.ds(j * 16, 16)] = v
        return 0

    lax.fori_loop(0, n_rows, body, 0)


def _sc_degree_body(ei3d, out, didx, ones_v, zbuf, acc):
    c = lax.axis_index("c")
    s = lax.axis_index("s")
    wid = c * NS + s
    pltpu.sync_copy(ei3d.at[1].at[wid], didx)
    _fill_rows(ones_v, CH, 16, 1.0)
    _fill_rows(zbuf, RPW, 16, 0.0)
    pltpu.sync_copy(zbuf.at[pl.ds(0, RPW)], acc.at[pl.ds(s * RPW, RPW)])

    @pl.when(s == NS - 1)
    def _():
        pltpu.sync_copy(zbuf.at[pl.ds(0, RTAIL)], acc.at[pl.ds(NS * RPW, RTAIL)])

    plsc.subcore_barrier()

    def body(j, _):
        pltpu.sync_copy(ones_v, acc.at[didx.at[j]], add=True)
        return 0

    lax.fori_loop(0, NCHUNK, body, 0)
    plsc.subcore_barrier()
    pltpu.sync_copy(acc.at[pl.ds(s * RPW, RPW)],
                    out.at[pl.ds(c * N + s * RPW, RPW)])

    @pl.when(s == NS - 1)
    def _():
        pltpu.sync_copy(acc.at[pl.ds(NS * RPW, N - NS * RPW)],
                        out.at[pl.ds(c * N + NS * RPW, N - NS * RPW)])


def _sc_aggregate_body(hs, ei3d, out, sidx, didx, rows0, rows1, zbuf,
                       acc, sem0, sem1):
    c = lax.axis_index("c")
    s = lax.axis_index("s")
    wid = c * NS + s
    hs64 = hs
    pltpu.sync_copy(ei3d.at[0].at[wid], sidx)
    pltpu.sync_copy(ei3d.at[1].at[wid], didx)
    _fill_rows(zbuf, RPW, H, 0.0)
    pltpu.sync_copy(zbuf.at[pl.ds(0, RPW)], acc.at[pl.ds(s * RPW, RPW)])

    @pl.when(s == NS - 1)
    def _():
        pltpu.sync_copy(zbuf.at[pl.ds(0, RTAIL)], acc.at[pl.ds(NS * RPW, RTAIL)])

    plsc.subcore_barrier()

    pltpu.async_copy(hs64.at[sidx.at[0]], rows0, sem0)

    def body(i, _):
        j0 = 2 * i
        pltpu.async_copy(hs64.at[sidx.at[j0 + 1]], rows1, sem1)
        pltpu.make_async_copy(hs64.at[sidx.at[j0]], rows0, sem0).wait()
        pltpu.sync_copy(rows0, acc.at[didx.at[j0]], add=True)
        pltpu.async_copy(hs64.at[sidx.at[j0 + 2]], rows0, sem0)
        pltpu.make_async_copy(hs64.at[sidx.at[j0 + 1]], rows1, sem1).wait()
        pltpu.sync_copy(rows1, acc.at[didx.at[j0 + 1]], add=True)
        return 0

    lax.fori_loop(0, (NCHUNK - 1) // 2, body, 0)
    pltpu.make_async_copy(hs64.at[sidx.at[NCHUNK - 1]], rows0, sem0).wait()
    pltpu.sync_copy(rows0, acc.at[didx.at[NCHUNK - 1]], add=True)
    plsc.subcore_barrier()
    pltpu.sync_copy(acc.at[pl.ds(s * RPW, RPW)],
                    out.at[pl.ds(c * N + s * RPW, RPW)])

    @pl.when(s == NS - 1)
    def _():
        pltpu.sync_copy(acc.at[pl.ds(NS * RPW, N - NS * RPW)],
                        out.at[pl.ds(c * N + NS * RPW, N - NS * RPW)])


@functools.cache
def _sc_kernels():
    mesh = plsc.VectorSubcoreMesh(core_axis_name="c", subcore_axis_name="s",
                                  num_cores=NC, num_subcores=NS)
    params = pltpu.CompilerParams(use_tc_tiling_on_sc=False)
    sc_degree = pl.kernel(
        _sc_degree_body,
        out_type=jax.ShapeDtypeStruct((2 * N, 16), jnp.float32),
        mesh=mesh,
        scratch_types=[
            pltpu.VMEM((NCHUNK, CH), jnp.int32),
            pltpu.VMEM((CH, 16), jnp.float32),
            pltpu.VMEM((RPW, 16), jnp.float32),
            pltpu.VMEM_SHARED((NA, 16), jnp.float32),
        ],
        compiler_params=params,
    )
    sc_aggregate = pl.kernel(
        _sc_aggregate_body,
        out_type=jax.ShapeDtypeStruct((2 * N, H), jnp.float32),
        mesh=mesh,
        scratch_types=[
            pltpu.VMEM((NCHUNK, CH), jnp.int32),
            pltpu.VMEM((NCHUNK, CH), jnp.int32),
            pltpu.VMEM((CH, H), jnp.float32),
            pltpu.VMEM((CH, H), jnp.float32),
            pltpu.VMEM((RPW, H), jnp.float32),
            pltpu.VMEM_SHARED((NA, H), jnp.float32),
            pltpu.SemaphoreType.DMA,
            pltpu.SemaphoreType.DMA,
        ],
        compiler_params=params,
    )
    return sc_degree, sc_aggregate


def _dis_from_degp(degp_ref):
    deg = 1.0 + degp_ref[pl.ds(0, N), pl.ds(0, 1)] + degp_ref[pl.ds(N, N), pl.ds(0, 1)]
    return lax.rsqrt(deg)


def _tc_head_body(x_ref, w1_ref, degp_ref, bcol_ref, hs_ref, se_ref):
    dis = _dis_from_degp(degp_ref)
    hs_ref[...] = jnp.dot(x_ref[...], w1_ref[...],
                          preferred_element_type=jnp.float32) * dis
    bcol = bcol_ref[...]
    gids = lax.broadcasted_iota(jnp.int32, (N, G), 1)
    starts = jnp.sum((bcol < gids).astype(jnp.float32), axis=0, keepdims=True)
    ends = jnp.sum((bcol <= gids).astype(jnp.float32), axis=0, keepdims=True)
    se_ref[...] = jnp.concatenate([starts, ends], axis=0).astype(jnp.int32)


def _tc_mid_body(p_ref, hsp_ref, degp_ref, w_ref, b_ref, hs_ref):
    dis = _dis_from_degp(degp_ref)
    agg = p_ref[pl.ds(0, N), :] + p_ref[pl.ds(N, N), :] + hsp_ref[...]
    h = jnp.maximum(agg * dis + b_ref[...], 0.0)
    hs_ref[...] = jnp.dot(h, w_ref[...], preferred_element_type=jnp.float32) * dis


NB = N // 16


def _tc_tail_body(p_ref, hsp_ref, degp_ref, b3_ref, brow_ref, bcol_ref,
                  batch8_ref, se_ref,
                  wc1_ref, bc1_ref, wc2_ref, bc2_ref, wc3_ref, bc3_ref,
                  out_ref, xmax_ref, h3_ref, bm_ref):
    dis = _dis_from_degp(degp_ref)
    agg = p_ref[pl.ds(0, N), :] + p_ref[pl.ds(N, N), :] + hsp_ref[...]
    h3_ref[...] = jnp.maximum(agg * dis + b3_ref[...], 0.0)
    h3 = h3_ref[...]

    gids = lax.broadcasted_iota(jnp.int32, (G, N), 0)
    sel = (brow_ref[...] == gids).astype(jnp.float32)
    counts = jnp.sum(sel, axis=1, keepdims=True)
    x_mean = jnp.dot(sel, h3, preferred_element_type=jnp.float32)
    x_mean = x_mean / jnp.maximum(counts, 1.0)

    neg = jnp.float32(-jnp.inf)
    bm_ref[...] = jnp.max(h3.reshape(NB, 16, H), axis=1)
    gmin = batch8_ref[:, pl.ds(0, 1)]
    gmax = batch8_ref[:, pl.ds(15, 1)]
    pb_id = jnp.where(gmin == gmax, gmin, -1)

    def mbody(g, _):
        pm = jnp.max(jnp.where(pb_id == g, bm_ref[...], neg),
                     axis=0, keepdims=True)
        s_g = se_ref[0, g]
        e_g = se_ref[1, g]
        b1 = jnp.minimum(s_g // 16, NB - 1)
        b2 = jnp.minimum(jnp.maximum(e_g - 1, 0) // 16, NB - 1)
        blk1 = h3_ref[pl.ds(b1 * 16, 16), :]
        msk1 = bcol_ref[pl.ds(b1 * 16, 16), :] == g
        m1 = jnp.max(jnp.where(msk1, blk1, neg), axis=0, keepdims=True)
        blk2 = h3_ref[pl.ds(b2 * 16, 16), :]
        msk2 = bcol_ref[pl.ds(b2 * 16, 16), :] == g
        m2 = jnp.max(jnp.where(msk2, blk2, neg), axis=0, keepdims=True)
        xmax_ref[pl.ds(g, 1), :] = jnp.maximum(pm, jnp.maximum(m1, m2))
        return 0

    lax.fori_loop(0, G, mbody, 0)
    xm = xmax_ref[...]
    x_max = jnp.where(jnp.isfinite(xm), xm, 0.0)

    gfeat = jnp.concatenate([x_mean, x_max], axis=1)
    g1 = jnp.maximum(jnp.dot(gfeat, wc1_ref[...],
                             preferred_element_type=jnp.float32) + bc1_ref[...], 0.0)
    g2 = jnp.maximum(jnp.dot(g1, wc2_ref[...],
                             preferred_element_type=jnp.float32) + bc2_ref[...], 0.0)
    logits = jnp.dot(g2, wc3_ref[...],
                     preferred_element_type=jnp.float32) + bc3_ref[...]
    m = jnp.max(logits, axis=1, keepdims=True)
    e = jnp.exp(logits - m)
    lse = jnp.log(jnp.sum(e, axis=1, keepdims=True))
    out_ref[...] = logits - m - lse


_tc_head = pl.pallas_call(
    _tc_head_body,
    out_shape=(jax.ShapeDtypeStruct((N, H), jnp.float32),
               jax.ShapeDtypeStruct((2, G), jnp.int32)),
)

_tc_mid = pl.pallas_call(
    _tc_mid_body,
    out_shape=jax.ShapeDtypeStruct((N, H), jnp.float32),
)

_tc_tail = pl.pallas_call(
    _tc_tail_body,
    out_shape=jax.ShapeDtypeStruct((G, 2), jnp.float32),
    in_specs=[pl.BlockSpec(memory_space=pltpu.VMEM)] * 7
    + [pl.BlockSpec(memory_space=pltpu.SMEM)]
    + [pl.BlockSpec(memory_space=pltpu.VMEM)] * 6,
    scratch_shapes=[pltpu.VMEM((G, H), jnp.float32),
                    pltpu.VMEM((N, H), jnp.float32),
                    pltpu.VMEM((NB, H), jnp.float32)],
)


def kernel(x, edge_index, batch, W1, b1, W2, b2, W3, b3,
           Wc1, bc1, Wc2, bc2, Wc3, bc3):
    pad = jnp.concatenate(
        [jnp.zeros((1, EPAD), jnp.int32),
         N + (jnp.arange(EPAD, dtype=jnp.int32) % 16).reshape(1, EPAD)], axis=0)
    ei3d = jnp.concatenate([edge_index, pad], axis=1).reshape(2, NW, NCHUNK, CH)
    brow = batch.reshape(1, N)
    bcol = batch.reshape(N, 1)
    batch16 = batch.reshape(NB, 16)

    sc_degree, sc_aggregate = _sc_kernels()
    degp = sc_degree(ei3d)
    hs1, se = _tc_head(x, W1, degp, bcol)
    p1 = sc_aggregate(hs1, ei3d)
    hs2 = _tc_mid(p1, hs1, degp, W2, b1.reshape(1, H))
    p2 = sc_aggregate(hs2, ei3d)
    hs3 = _tc_mid(p2, hs2, degp, W3, b2.reshape(1, H))
    p3 = sc_aggregate(hs3, ei3d)
    return _tc_tail(p3, hs3, degp, b3.reshape(1, H), brow, bcol, batch16, se,
                    Wc1, bc1.reshape(1, H), Wc2, bc2.reshape(1, H // 2),
                    Wc3, bc3.reshape(1, 2))


# speedup vs baseline: 1.5704x; 1.5704x over previous
import functools
import jax
import jax.numpy as jnp
from jax import lax
from jax.experimental import pallas as pl
from jax.experimental.pallas import tpu as pltpu
from jax.experimental.pallas import tpu_sc as plsc

N = 10000
E = 320000
H = 64
G = 128
NC = 2
NS = 16
NW = NC * NS
CH = 128
NCHUNK = -(-E // (NW * CH))
EPAD = NW * CH * NCHUNK - E
NA = N + 16
RPW = 624
RTAIL = NA - NS * RPW

def _fill_rows(ref, n_rows, n_cols, value):
    v = jnp.full((16,), value, dtype=ref.dtype)

    def body(i, _):
        for j in range(n_cols // 16):
            ref[i, pl.ds(j * 16, 16)] = v
        return 0

    lax.fori_loop(0, n_rows, body, 0)


def _sc_degree_body(ei3d, out, didx, ones_v, zbuf, acc):
    c = lax.axis_index("c")
    s = lax.axis_index("s")
    wid = c * NS + s
    pltpu.sync_copy(ei3d.at[1].at[wid], didx)
    _fill_rows(ones_v, CH, 16, 1.0)
    _fill_rows(zbuf, RPW, 16, 0.0)
    pltpu.sync_copy(zbuf.at[pl.ds(0, RPW)], acc.at[pl.ds(s * RPW, RPW)])

    @pl.when(s == NS - 1)
    def _():
        pltpu.sync_copy(zbuf.at[pl.ds(0, RTAIL)], acc.at[pl.ds(NS * RPW, RTAIL)])

    plsc.subcore_barrier()

    def body(j, _):
        pltpu.sync_copy(ones_v, acc.at[didx.at[j]], add=True)
        return 0

    lax.fori_loop(0, NCHUNK, body, 0)
    plsc.subcore_barrier()
    pltpu.sync_copy(acc.at[pl.ds(s * RPW, RPW)],
                    out.at[pl.ds(c * N + s * RPW, RPW)])

    @pl.when(s == NS - 1)
    def _():
        pltpu.sync_copy(acc.at[pl.ds(NS * RPW, N - NS * RPW)],
                        out.at[pl.ds(c * N + NS * RPW, N - NS * RPW)])


def _sc_aggregate_body(hs, ei3d, out, sidx, didx, rows0, rows1, zbuf,
                       acc, sem0, sem1):
    c = lax.axis_index("c")
    s = lax.axis_index("s")
    wid = c * NS + s
    hs64 = hs
    pltpu.sync_copy(ei3d.at[0].at[wid], sidx)
    pltpu.sync_copy(ei3d.at[1].at[wid], didx)
    _fill_rows(zbuf, RPW, H, 0.0)
    pltpu.sync_copy(zbuf.at[pl.ds(0, RPW)], acc.at[pl.ds(s * RPW, RPW)])

    @pl.when(s == NS - 1)
    def _():
        pltpu.sync_copy(zbuf.at[pl.ds(0, RTAIL)], acc.at[pl.ds(NS * RPW, RTAIL)])

    plsc.subcore_barrier()

    pltpu.async_copy(hs64.at[sidx.at[0]], rows0, sem0)

    def body(i, _):
        j0 = 2 * i
        pltpu.async_copy(hs64.at[sidx.at[j0 + 1]], rows1, sem1)
        pltpu.make_async_copy(hs64.at[sidx.at[j0]], rows0, sem0).wait()
        pltpu.sync_copy(rows0, acc.at[didx.at[j0]], add=True)
        pltpu.async_copy(hs64.at[sidx.at[j0 + 2]], rows0, sem0)
        pltpu.make_async_copy(hs64.at[sidx.at[j0 + 1]], rows1, sem1).wait()
        pltpu.sync_copy(rows1, acc.at[didx.at[j0 + 1]], add=True)
        return 0

    lax.fori_loop(0, (NCHUNK - 1) // 2, body, 0)
    pltpu.make_async_copy(hs64.at[sidx.at[NCHUNK - 1]], rows0, sem0).wait()
    pltpu.sync_copy(rows0, acc.at[didx.at[NCHUNK - 1]], add=True)
    plsc.subcore_barrier()
    pltpu.sync_copy(acc.at[pl.ds(s * RPW, RPW)],
                    out.at[pl.ds(c * N + s * RPW, RPW)])

    @pl.when(s == NS - 1)
    def _():
        pltpu.sync_copy(acc.at[pl.ds(NS * RPW, N - NS * RPW)],
                        out.at[pl.ds(c * N + NS * RPW, N - NS * RPW)])


@functools.cache
def _sc_kernels():
    mesh = plsc.VectorSubcoreMesh(core_axis_name="c", subcore_axis_name="s",
                                  num_cores=NC, num_subcores=NS)
    params = pltpu.CompilerParams(use_tc_tiling_on_sc=False)
    sc_degree = pl.kernel(
        _sc_degree_body,
        out_type=jax.ShapeDtypeStruct((2 * N, 16), jnp.float32),
        mesh=mesh,
        scratch_types=[
            pltpu.VMEM((NCHUNK, CH), jnp.int32),
            pltpu.VMEM((CH, 16), jnp.float32),
            pltpu.VMEM((RPW, 16), jnp.float32),
            pltpu.VMEM_SHARED((NA, 16), jnp.float32),
        ],
        compiler_params=params,
    )
    sc_aggregate = pl.kernel(
        _sc_aggregate_body,
        out_type=jax.ShapeDtypeStruct((2 * N, H), jnp.float32),
        mesh=mesh,
        scratch_types=[
            pltpu.VMEM((NCHUNK, CH), jnp.int32),
            pltpu.VMEM((NCHUNK, CH), jnp.int32),
            pltpu.VMEM((CH, H), jnp.float32),
            pltpu.VMEM((CH, H), jnp.float32),
            pltpu.VMEM((RPW, H), jnp.float32),
            pltpu.VMEM_SHARED((NA, H), jnp.float32),
            pltpu.SemaphoreType.DMA,
            pltpu.SemaphoreType.DMA,
        ],
        compiler_params=params,
    )
    return sc_degree, sc_aggregate


def _dis_from_degp(degp_ref):
    deg = 1.0 + degp_ref[pl.ds(0, N), pl.ds(0, 1)] + degp_ref[pl.ds(N, N), pl.ds(0, 1)]
    return lax.rsqrt(deg)


def _tc_head_body(x_ref, w1_ref, degp_ref, bcol_ref, hs_ref, se_ref):
    dis = _dis_from_degp(degp_ref)
    hs_ref[...] = jnp.dot(x_ref[...], w1_ref[...],
                          preferred_element_type=jnp.float32) * dis
    bcol = bcol_ref[...]
    gids = lax.broadcasted_iota(jnp.int32, (N, G), 1)
    starts = jnp.sum((bcol < gids).astype(jnp.float32), axis=0, keepdims=True)
    ends = jnp.sum((bcol <= gids).astype(jnp.float32), axis=0, keepdims=True)
    se_ref[...] = jnp.concatenate([starts, ends], axis=0).astype(jnp.int32)


def _tc_mid_body(p_ref, hsp_ref, degp_ref, w_ref, b_ref, hs_ref):
    dis = _dis_from_degp(degp_ref)
    agg = p_ref[pl.ds(0, N), :] + p_ref[pl.ds(N, N), :] + hsp_ref[...]
    h = jnp.maximum(agg * dis + b_ref[...], 0.0)
    hs_ref[...] = jnp.dot(h, w_ref[...], preferred_element_type=jnp.float32) * dis


NB = N // 16


def _tc_tail_body(p_ref, hsp_ref, degp_ref, b3_ref, brow_ref, bcol_ref,
                  batch8_ref, se_ref,
                  wc1_ref, bc1_ref, wc2_ref, bc2_ref, wc3_ref, bc3_ref,
                  out_ref, xmax_ref, h3_ref, bm_ref):
    dis = _dis_from_degp(degp_ref)
    agg = p_ref[pl.ds(0, N), :] + p_ref[pl.ds(N, N), :] + hsp_ref[...]
    h3_ref[...] = jnp.maximum(agg * dis + b3_ref[...], 0.0)
    h3 = h3_ref[...]

    gids = lax.broadcasted_iota(jnp.int32, (G, N), 0)
    sel = (brow_ref[...] == gids).astype(jnp.float32)
    counts = jnp.sum(sel, axis=1, keepdims=True)
    x_mean = jnp.dot(sel, h3, preferred_element_type=jnp.float32)
    x_mean = x_mean / jnp.maximum(counts, 1.0)

    neg = jnp.float32(-jnp.inf)
    bm_ref[...] = jnp.max(h3.reshape(NB, 16, H), axis=1)
    gmin = batch8_ref[:, pl.ds(0, 1)]
    gmax = batch8_ref[:, pl.ds(15, 1)]
    pb_id = jnp.where(gmin == gmax, gmin, -1)

    def mbody(g, _):
        pm = jnp.max(jnp.where(pb_id == g, bm_ref[...], neg),
                     axis=0, keepdims=True)
        s_g = se_ref[0, g]
        e_g = se_ref[1, g]
        b1 = jnp.minimum(s_g // 16, NB - 1)
        b2 = jnp.minimum(jnp.maximum(e_g - 1, 0) // 16, NB - 1)
        blk1 = h3_ref[pl.ds(b1 * 16, 16), :]
        msk1 = bcol_ref[pl.ds(b1 * 16, 16), :] == g
        m1 = jnp.max(jnp.where(msk1, blk1, neg), axis=0, keepdims=True)
        blk2 = h3_ref[pl.ds(b2 * 16, 16), :]
        msk2 = bcol_ref[pl.ds(b2 * 16, 16), :] == g
        m2 = jnp.max(jnp.where(msk2, blk2, neg), axis=0, keepdims=True)
        xmax_ref[pl.ds(g, 1), :] = jnp.maximum(pm, jnp.maximum(m1, m2))
        return 0

    lax.fori_loop(0, G, mbody, 0)
    xm = xmax_ref[...]
    x_max = jnp.where(jnp.isfinite(xm), xm, 0.0)

    gfeat = jnp.concatenate([x_mean, x_max], axis=1)
    g1 = jnp.maximum(jnp.dot(gfeat, wc1_ref[...],
                             preferred_element_type=jnp.float32) + bc1_ref[...], 0.0)
    g2 = jnp.maximum(jnp.dot(g1, wc2_ref[...],
                             preferred_element_type=jnp.float32) + bc2_ref[...], 0.0)
    logits = jnp.dot(g2, wc3_ref[...],
                     preferred_element_type=jnp.float32) + bc3_ref[...]
    m = jnp.max(logits, axis=1, keepdims=True)
    e = jnp.exp(logits - m)
    lse = jnp.log(jnp.sum(e, axis=1, keepdims=True))
    out_ref[...] = logits - m - lse


_tc_head = pl.pallas_call(
    _tc_head_body,
    out_shape=(jax.ShapeDtypeStruct((N, H), jnp.float32),
               jax.ShapeDtypeStruct((2, G), jnp.int32)),
)

_tc_mid = pl.pallas_call(
    _tc_mid_body,
    out_shape=jax.ShapeDtypeStruct((N, H), jnp.float32),
)

_tc_tail = pl.pallas_call(
    _tc_tail_body,
    out_shape=jax.ShapeDtypeStruct((G, 2), jnp.float32),
    in_specs=[pl.BlockSpec(memory_space=pltpu.VMEM)] * 7
    + [pl.BlockSpec(memory_space=pltpu.SMEM)]
    + [pl.BlockSpec(memory_space=pltpu.VMEM)] * 6,
    scratch_shapes=[pltpu.VMEM((G, H), jnp.float32),
                    pltpu.VMEM((N, H), jnp.float32),
                    pltpu.VMEM((NB, H), jnp.float32)],
)


def kernel(x, edge_index, batch, W1, b1, W2, b2, W3, b3,
           Wc1, bc1, Wc2, bc2, Wc3, bc3):
    pidx = jnp.arange(EPAD, dtype=jnp.int32)
    pad = jnp.concatenate(
        [(pidx * 8 % N).reshape(1, EPAD),
         (N + pidx % 16).reshape(1, EPAD)], axis=0)
    ei3d = jnp.concatenate([edge_index, pad], axis=1).reshape(2, NW, NCHUNK, CH)
    brow = batch.reshape(1, N)
    bcol = batch.reshape(N, 1)
    batch16 = batch.reshape(NB, 16)

    sc_degree, sc_aggregate = _sc_kernels()
    degp = sc_degree(ei3d)
    hs1, se = _tc_head(x, W1, degp, bcol)
    p1 = sc_aggregate(hs1, ei3d)
    hs2 = _tc_mid(p1, hs1, degp, W2, b1.reshape(1, H))
    p2 = sc_aggregate(hs2, ei3d)
    hs3 = _tc_mid(p2, hs2, degp, W3, b2.reshape(1, H))
    p3 = sc_aggregate(hs3, ei3d)
    return _tc_tail(p3, hs3, degp, b3.reshape(1, H), brow, bcol, batch16, se,
                    Wc1, bc1.reshape(1, H), Wc2, bc2.reshape(1, H // 2),
                    Wc3, bc3.reshape(1, 2))


# speedup vs baseline: 1.6964x; 1.0802x over previous
import functools
import jax
import jax.numpy as jnp
from jax import lax
from jax.experimental import pallas as pl
from jax.experimental.pallas import tpu as pltpu
from jax.experimental.pallas import tpu_sc as plsc

N = 10000
E = 320000
H = 64
G = 128
NC = 2
NS = 16
NW = NC * NS
CH = 128
NCHUNK = -(-E // (NW * CH))
EPAD = NW * CH * NCHUNK - E
NA = N + 16
RPW = 624
RTAIL = NA - NS * RPW

def _fill_rows(ref, n_rows, n_cols, value):
    v = jnp.full((16,), value, dtype=ref.dtype)

    def body(i, _):
        for j in range(n_cols // 16):
            ref[i, pl.ds(j * 16, 16)] = v
        return 0

    lax.fori_loop(0, n_rows, body, 0)


def _sc_degree_body(ei3d, out, didx, ones_v, zbuf, acc):
    c = lax.axis_index("c")
    s = lax.axis_index("s")
    wid = c * NS + s
    pltpu.sync_copy(ei3d.at[1].at[wid], didx)
    _fill_rows(ones_v, CH, 16, 1.0)
    _fill_rows(zbuf, RPW, 16, 0.0)
    pltpu.sync_copy(zbuf.at[pl.ds(0, RPW)], acc.at[pl.ds(s * RPW, RPW)])

    @pl.when(s == NS - 1)
    def _():
        pltpu.sync_copy(zbuf.at[pl.ds(0, RTAIL)], acc.at[pl.ds(NS * RPW, RTAIL)])

    plsc.subcore_barrier()

    def body(j, _):
        pltpu.sync_copy(ones_v, acc.at[didx.at[j]], add=True)
        return 0

    lax.fori_loop(0, NCHUNK, body, 0)
    plsc.subcore_barrier()
    pltpu.sync_copy(acc.at[pl.ds(s * RPW, RPW)],
                    out.at[pl.ds(c * N + s * RPW, RPW)])

    @pl.when(s == NS - 1)
    def _():
        pltpu.sync_copy(acc.at[pl.ds(NS * RPW, N - NS * RPW)],
                        out.at[pl.ds(c * N + NS * RPW, N - NS * RPW)])


def _sc_aggregate_body(hs, ei3d, out, sidx, didx, rows0, rows1, zbuf,
                       acc, sem0, sem1):
    c = lax.axis_index("c")
    s = lax.axis_index("s")
    wid = c * NS + s
    hs64 = hs
    pltpu.sync_copy(ei3d.at[0].at[wid], sidx)
    pltpu.sync_copy(ei3d.at[1].at[wid], didx)
    _fill_rows(zbuf, RPW, H, 0.0)
    pltpu.sync_copy(zbuf.at[pl.ds(0, RPW)], acc.at[pl.ds(s * RPW, RPW)])

    @pl.when(s == NS - 1)
    def _():
        pltpu.sync_copy(zbuf.at[pl.ds(0, RTAIL)], acc.at[pl.ds(NS * RPW, RTAIL)])

    plsc.subcore_barrier()

    pltpu.async_copy(hs64.at[sidx.at[0]], rows0, sem0)

    def body(i, _):
        j0 = 2 * i
        pltpu.async_copy(hs64.at[sidx.at[j0 + 1]], rows1, sem1)
        pltpu.make_async_copy(hs64.at[sidx.at[j0]], rows0, sem0).wait()
        pltpu.sync_copy(rows0, acc.at[didx.at[j0]], add=True)
        pltpu.async_copy(hs64.at[sidx.at[j0 + 2]], rows0, sem0)
        pltpu.make_async_copy(hs64.at[sidx.at[j0 + 1]], rows1, sem1).wait()
        pltpu.sync_copy(rows1, acc.at[didx.at[j0 + 1]], add=True)
        return 0

    lax.fori_loop(0, (NCHUNK - 1) // 2, body, 0)
    pltpu.make_async_copy(hs64.at[sidx.at[NCHUNK - 1]], rows0, sem0).wait()
    pltpu.sync_copy(rows0, acc.at[didx.at[NCHUNK - 1]], add=True)
    plsc.subcore_barrier()
    pltpu.sync_copy(acc.at[pl.ds(s * RPW, RPW)],
                    out.at[pl.ds(s * RPW, RPW), pl.ds(c * H, H)])

    @pl.when(s == NS - 1)
    def _():
        pltpu.sync_copy(acc.at[pl.ds(NS * RPW, N - NS * RPW)],
                        out.at[pl.ds(NS * RPW, N - NS * RPW), pl.ds(c * H, H)])


@functools.cache
def _sc_kernels():
    mesh = plsc.VectorSubcoreMesh(core_axis_name="c", subcore_axis_name="s",
                                  num_cores=NC, num_subcores=NS)
    params = pltpu.CompilerParams(use_tc_tiling_on_sc=False)
    sc_degree = pl.kernel(
        _sc_degree_body,
        out_type=jax.ShapeDtypeStruct((2 * N, 16), jnp.float32),
        mesh=mesh,
        scratch_types=[
            pltpu.VMEM((NCHUNK, CH), jnp.int32),
            pltpu.VMEM((CH, 16), jnp.float32),
            pltpu.VMEM((RPW, 16), jnp.float32),
            pltpu.VMEM_SHARED((NA, 16), jnp.float32),
        ],
        compiler_params=params,
    )
    sc_aggregate = pl.kernel(
        _sc_aggregate_body,
        out_type=jax.ShapeDtypeStruct((N, 2 * H), jnp.float32),
        mesh=mesh,
        scratch_types=[
            pltpu.VMEM((NCHUNK, CH), jnp.int32),
            pltpu.VMEM((NCHUNK, CH), jnp.int32),
            pltpu.VMEM((CH, H), jnp.float32),
            pltpu.VMEM((CH, H), jnp.float32),
            pltpu.VMEM((RPW, H), jnp.float32),
            pltpu.VMEM_SHARED((NA, H), jnp.float32),
            pltpu.SemaphoreType.DMA,
            pltpu.SemaphoreType.DMA,
        ],
        compiler_params=params,
    )
    return sc_degree, sc_aggregate


def _dis_from_degp(degp_ref):
    deg = 1.0 + degp_ref[pl.ds(0, N), pl.ds(0, 1)] + degp_ref[pl.ds(N, N), pl.ds(0, 1)]
    return lax.rsqrt(deg)


def _tc_head_body(x_ref, w1_ref, degp_ref, bcol_ref, hs_ref, se_ref):
    dis = _dis_from_degp(degp_ref)
    hs_ref[...] = jnp.dot(x_ref[...], w1_ref[...],
                          preferred_element_type=jnp.float32) * dis
    bcol = bcol_ref[...]
    gids = lax.broadcasted_iota(jnp.int32, (N, G), 1)
    starts = jnp.sum((bcol < gids).astype(jnp.float32), axis=0, keepdims=True)
    ends = jnp.sum((bcol <= gids).astype(jnp.float32), axis=0, keepdims=True)
    se_ref[...] = jnp.concatenate([starts, ends], axis=0).astype(jnp.int32)


def _tc_mid_body(p_ref, hsp_ref, degp_ref, w_ref, b_ref, hs_ref):
    dis = _dis_from_degp(degp_ref)
    agg = (p_ref[:, pl.ds(0, H)] + p_ref[:, pl.ds(H, H)] + hsp_ref[...])
    h = jnp.maximum(agg * dis + b_ref[...], 0.0)
    hs_ref[...] = jnp.dot(h, w_ref[...], preferred_element_type=jnp.float32) * dis


NB = N // 16


def _tc_tail_body(p_ref, hsp_ref, degp_ref, b3_ref, brow_ref, bcol_ref,
                  batch8_ref, se_ref,
                  wc1_ref, bc1_ref, wc2_ref, bc2_ref, wc3_ref, bc3_ref,
                  out_ref, xmax_ref, h3_ref, bm_ref):
    dis = _dis_from_degp(degp_ref)
    agg = (p_ref[:, pl.ds(0, H)] + p_ref[:, pl.ds(H, H)] + hsp_ref[...])
    h3_ref[...] = jnp.maximum(agg * dis + b3_ref[...], 0.0)
    h3 = h3_ref[...]

    gids = lax.broadcasted_iota(jnp.int32, (G, N), 0)
    sel = (brow_ref[...] == gids).astype(jnp.float32)
    counts = jnp.sum(sel, axis=1, keepdims=True)
    x_mean = jnp.dot(sel, h3, preferred_element_type=jnp.float32)
    x_mean = x_mean / jnp.maximum(counts, 1.0)

    neg = jnp.float32(-jnp.inf)
    bm_ref[...] = jnp.max(h3.reshape(NB, 16, H), axis=1)
    gmin = batch8_ref[:, pl.ds(0, 1)]
    gmax = batch8_ref[:, pl.ds(15, 1)]
    pb_id = jnp.where(gmin == gmax, gmin, -1)

    def mbody(g, _):
        pm = jnp.max(jnp.where(pb_id == g, bm_ref[...], neg),
                     axis=0, keepdims=True)
        s_g = se_ref[0, g]
        e_g = se_ref[1, g]
        b1 = jnp.minimum(s_g // 16, NB - 1)
        b2 = jnp.minimum(jnp.maximum(e_g - 1, 0) // 16, NB - 1)
        blk1 = h3_ref[pl.ds(b1 * 16, 16), :]
        msk1 = bcol_ref[pl.ds(b1 * 16, 16), :] == g
        m1 = jnp.max(jnp.where(msk1, blk1, neg), axis=0, keepdims=True)
        blk2 = h3_ref[pl.ds(b2 * 16, 16), :]
        msk2 = bcol_ref[pl.ds(b2 * 16, 16), :] == g
        m2 = jnp.max(jnp.where(msk2, blk2, neg), axis=0, keepdims=True)
        xmax_ref[pl.ds(g, 1), :] = jnp.maximum(pm, jnp.maximum(m1, m2))
        return 0

    lax.fori_loop(0, G, mbody, 0)
    xm = xmax_ref[...]
    x_max = jnp.where(jnp.isfinite(xm), xm, 0.0)

    gfeat = jnp.concatenate([x_mean, x_max], axis=1)
    g1 = jnp.maximum(jnp.dot(gfeat, wc1_ref[...],
                             preferred_element_type=jnp.float32) + bc1_ref[...], 0.0)
    g2 = jnp.maximum(jnp.dot(g1, wc2_ref[...],
                             preferred_element_type=jnp.float32) + bc2_ref[...], 0.0)
    logits = jnp.dot(g2, wc3_ref[...],
                     preferred_element_type=jnp.float32) + bc3_ref[...]
    m = jnp.max(logits, axis=1, keepdims=True)
    e = jnp.exp(logits - m)
    lse = jnp.log(jnp.sum(e, axis=1, keepdims=True))
    out_ref[...] = logits - m - lse


_tc_head = pl.pallas_call(
    _tc_head_body,
    out_shape=(jax.ShapeDtypeStruct((N, H), jnp.float32),
               jax.ShapeDtypeStruct((2, G), jnp.int32)),
)

_tc_mid = pl.pallas_call(
    _tc_mid_body,
    out_shape=jax.ShapeDtypeStruct((N, H), jnp.float32),
)

_tc_tail = pl.pallas_call(
    _tc_tail_body,
    out_shape=jax.ShapeDtypeStruct((G, 2), jnp.float32),
    in_specs=[pl.BlockSpec(memory_space=pltpu.VMEM)] * 7
    + [pl.BlockSpec(memory_space=pltpu.SMEM)]
    + [pl.BlockSpec(memory_space=pltpu.VMEM)] * 6,
    scratch_shapes=[pltpu.VMEM((G, H), jnp.float32),
                    pltpu.VMEM((N, H), jnp.float32),
                    pltpu.VMEM((NB, H), jnp.float32)],
)


def kernel(x, edge_index, batch, W1, b1, W2, b2, W3, b3,
           Wc1, bc1, Wc2, bc2, Wc3, bc3):
    pidx = jnp.arange(EPAD, dtype=jnp.int32)
    pad = jnp.concatenate(
        [(pidx * 8 % N).reshape(1, EPAD),
         (N + pidx % 16).reshape(1, EPAD)], axis=0)
    ei3d = jnp.concatenate([edge_index, pad], axis=1).reshape(2, NW, NCHUNK, CH)
    brow = batch.reshape(1, N)
    bcol = batch.reshape(N, 1)
    batch16 = batch.reshape(NB, 16)

    sc_degree, sc_aggregate = _sc_kernels()
    degp = sc_degree(ei3d)
    hs1, se = _tc_head(x, W1, degp, bcol)
    p1 = sc_aggregate(hs1, ei3d)
    hs2 = _tc_mid(p1, hs1, degp, W2, b1.reshape(1, H))
    p2 = sc_aggregate(hs2, ei3d)
    hs3 = _tc_mid(p2, hs2, degp, W3, b2.reshape(1, H))
    p3 = sc_aggregate(hs3, ei3d)
    return _tc_tail(p3, hs3, degp, b3.reshape(1, H), brow, bcol, batch16, se,
                    Wc1, bc1.reshape(1, H), Wc2, bc2.reshape(1, H // 2),
                    Wc3, bc3.reshape(1, 2))


# speedup vs baseline: 1.7196x; 1.0137x over previous
import functools
import jax
import jax.numpy as jnp
from jax import lax
from jax.experimental import pallas as pl
from jax.experimental.pallas import tpu as pltpu
from jax.experimental.pallas import tpu_sc as plsc

N = 10000
E = 320000
H = 64
G = 128
NC = 2
NS = 16
NW = NC * NS
CH = 128
NCHUNK = -(-E // (NW * CH))
EPAD = NW * CH * NCHUNK - E
NA = N + 16
RPW = 624
RTAIL = NA - NS * RPW

def _fill_rows(ref, n_rows, n_cols, value):
    v = jnp.full((16,), value, dtype=ref.dtype)

    def body(i, _):
        for j in range(n_cols // 16):
            ref[i, pl.ds(j * 16, 16)] = v
        return 0

    lax.fori_loop(0, n_rows, body, 0)


def _sc_degree_body(ei3d, out, didx, ones_v, zbuf, acc):
    c = lax.axis_index("c")
    s = lax.axis_index("s")
    wid = c * NS + s
    pltpu.sync_copy(ei3d.at[1].at[wid], didx)
    _fill_rows(ones_v, CH, 16, 1.0)
    _fill_rows(zbuf, RPW, 16, 0.0)
    pltpu.sync_copy(zbuf.at[pl.ds(0, RPW)], acc.at[pl.ds(s * RPW, RPW)])

    @pl.when(s == NS - 1)
    def _():
        pltpu.sync_copy(zbuf.at[pl.ds(0, RTAIL)], acc.at[pl.ds(NS * RPW, RTAIL)])

    plsc.subcore_barrier()

    def body(j, _):
        pltpu.sync_copy(ones_v, acc.at[didx.at[j]], add=True)
        return 0

    lax.fori_loop(0, NCHUNK, body, 0)
    plsc.subcore_barrier()
    pltpu.sync_copy(acc.at[pl.ds(s * RPW, RPW)],
                    out.at[pl.ds(s * RPW, RPW), pl.ds(c * 16, 16)])

    @pl.when(s == NS - 1)
    def _():
        pltpu.sync_copy(acc.at[pl.ds(NS * RPW, N - NS * RPW)],
                        out.at[pl.ds(NS * RPW, N - NS * RPW), pl.ds(c * 16, 16)])


def _sc_aggregate_body(hs, ei3d, out, sidx, didx, rows0, rows1, zbuf,
                       acc, sem0, sem1):
    c = lax.axis_index("c")
    s = lax.axis_index("s")
    wid = c * NS + s
    hs64 = hs
    pltpu.sync_copy(ei3d.at[0].at[wid], sidx)
    pltpu.sync_copy(ei3d.at[1].at[wid], didx)
    _fill_rows(zbuf, RPW, H, 0.0)
    pltpu.sync_copy(zbuf.at[pl.ds(0, RPW)], acc.at[pl.ds(s * RPW, RPW)])

    @pl.when(s == NS - 1)
    def _():
        pltpu.sync_copy(zbuf.at[pl.ds(0, RTAIL)], acc.at[pl.ds(NS * RPW, RTAIL)])

    plsc.subcore_barrier()

    pltpu.async_copy(hs64.at[sidx.at[0]], rows0, sem0)

    def body(i, _):
        j0 = 2 * i
        pltpu.async_copy(hs64.at[sidx.at[j0 + 1]], rows1, sem1)
        pltpu.make_async_copy(hs64.at[sidx.at[j0]], rows0, sem0).wait()
        pltpu.sync_copy(rows0, acc.at[didx.at[j0]], add=True)
        pltpu.async_copy(hs64.at[sidx.at[j0 + 2]], rows0, sem0)
        pltpu.make_async_copy(hs64.at[sidx.at[j0 + 1]], rows1, sem1).wait()
        pltpu.sync_copy(rows1, acc.at[didx.at[j0 + 1]], add=True)
        return 0

    lax.fori_loop(0, (NCHUNK - 1) // 2, body, 0)
    pltpu.make_async_copy(hs64.at[sidx.at[NCHUNK - 1]], rows0, sem0).wait()
    pltpu.sync_copy(rows0, acc.at[didx.at[NCHUNK - 1]], add=True)
    plsc.subcore_barrier()
    pltpu.sync_copy(acc.at[pl.ds(s * RPW, RPW)],
                    out.at[pl.ds(s * RPW, RPW), pl.ds(c * H, H)])

    @pl.when(s == NS - 1)
    def _():
        pltpu.sync_copy(acc.at[pl.ds(NS * RPW, N - NS * RPW)],
                        out.at[pl.ds(NS * RPW, N - NS * RPW), pl.ds(c * H, H)])


@functools.cache
def _sc_kernels():
    mesh = plsc.VectorSubcoreMesh(core_axis_name="c", subcore_axis_name="s",
                                  num_cores=NC, num_subcores=NS)
    params = pltpu.CompilerParams(use_tc_tiling_on_sc=False)
    sc_degree = pl.kernel(
        _sc_degree_body,
        out_type=jax.ShapeDtypeStruct((N, 128), jnp.float32),
        mesh=mesh,
        scratch_types=[
            pltpu.VMEM((NCHUNK, CH), jnp.int32),
            pltpu.VMEM((CH, 16), jnp.float32),
            pltpu.VMEM((RPW, 16), jnp.float32),
            pltpu.VMEM_SHARED((NA, 16), jnp.float32),
        ],
        compiler_params=params,
    )
    sc_aggregate = pl.kernel(
        _sc_aggregate_body,
        out_type=jax.ShapeDtypeStruct((N, 2 * H), jnp.float32),
        mesh=mesh,
        scratch_types=[
            pltpu.VMEM((NCHUNK, CH), jnp.int32),
            pltpu.VMEM((NCHUNK, CH), jnp.int32),
            pltpu.VMEM((CH, H), jnp.float32),
            pltpu.VMEM((CH, H), jnp.float32),
            pltpu.VMEM((RPW, H), jnp.float32),
            pltpu.VMEM_SHARED((NA, H), jnp.float32),
            pltpu.SemaphoreType.DMA,
            pltpu.SemaphoreType.DMA,
        ],
        compiler_params=params,
    )
    return sc_degree, sc_aggregate


def _dis_from_degp(degp_ref):
    deg = 1.0 + degp_ref[:, pl.ds(0, 1)] + degp_ref[:, pl.ds(16, 1)]
    return lax.rsqrt(deg)


def _tc_head_body(x_ref, w1_ref, degp_ref, bcol_ref, hs_ref, se_ref):
    dis = _dis_from_degp(degp_ref)
    hs_ref[...] = jnp.dot(x_ref[...], w1_ref[...],
                          preferred_element_type=jnp.float32) * dis
    bcol = bcol_ref[...]
    gids = lax.broadcasted_iota(jnp.int32, (N, G), 1)
    starts = jnp.sum((bcol < gids).astype(jnp.float32), axis=0, keepdims=True)
    ends = jnp.sum((bcol <= gids).astype(jnp.float32), axis=0, keepdims=True)
    se_ref[...] = jnp.concatenate([starts, ends], axis=0).astype(jnp.int32)


def _tc_mid_body(p_ref, hsp_ref, degp_ref, w_ref, b_ref, hs_ref):
    dis = _dis_from_degp(degp_ref)
    agg = (p_ref[:, pl.ds(0, H)] + p_ref[:, pl.ds(H, H)] + hsp_ref[...])
    h = jnp.maximum(agg * dis + b_ref[...], 0.0)
    hs_ref[...] = jnp.dot(h, w_ref[...], preferred_element_type=jnp.float32) * dis


NB = N // 16


def _tc_tail_body(p_ref, hsp_ref, degp_ref, b3_ref, brow_ref, bcol_ref,
                  batch8_ref, se_ref,
                  wc1_ref, bc1_ref, wc2_ref, bc2_ref, wc3_ref, bc3_ref,
                  out_ref, xmax_ref, h3_ref, bm_ref):
    dis = _dis_from_degp(degp_ref)
    agg = (p_ref[:, pl.ds(0, H)] + p_ref[:, pl.ds(H, H)] + hsp_ref[...])
    h3_ref[...] = jnp.maximum(agg * dis + b3_ref[...], 0.0)
    h3 = h3_ref[...]

    gids = lax.broadcasted_iota(jnp.int32, (G, N), 0)
    sel = (brow_ref[...] == gids).astype(jnp.float32)
    counts = jnp.sum(sel, axis=1, keepdims=True)
    x_mean = jnp.dot(sel, h3, preferred_element_type=jnp.float32)
    x_mean = x_mean / jnp.maximum(counts, 1.0)

    neg = jnp.float32(-jnp.inf)
    bm_ref[...] = jnp.max(h3.reshape(NB, 16, H), axis=1)
    gmin = batch8_ref[:, pl.ds(0, 1)]
    gmax = batch8_ref[:, pl.ds(15, 1)]
    pb_id = jnp.where(gmin == gmax, gmin, -1)

    def mbody(g, _):
        pm = jnp.max(jnp.where(pb_id == g, bm_ref[...], neg),
                     axis=0, keepdims=True)
        s_g = se_ref[0, g]
        e_g = se_ref[1, g]
        b1 = jnp.minimum(s_g // 16, NB - 1)
        b2 = jnp.minimum(jnp.maximum(e_g - 1, 0) // 16, NB - 1)
        blk1 = h3_ref[pl.ds(b1 * 16, 16), :]
        msk1 = bcol_ref[pl.ds(b1 * 16, 16), :] == g
        m1 = jnp.max(jnp.where(msk1, blk1, neg), axis=0, keepdims=True)
        blk2 = h3_ref[pl.ds(b2 * 16, 16), :]
        msk2 = bcol_ref[pl.ds(b2 * 16, 16), :] == g
        m2 = jnp.max(jnp.where(msk2, blk2, neg), axis=0, keepdims=True)
        xmax_ref[pl.ds(g, 1), :] = jnp.maximum(pm, jnp.maximum(m1, m2))
        return 0

    lax.fori_loop(0, G, mbody, 0)
    xm = xmax_ref[...]
    x_max = jnp.where(jnp.isfinite(xm), xm, 0.0)

    gfeat = jnp.concatenate([x_mean, x_max], axis=1)
    g1 = jnp.maximum(jnp.dot(gfeat, wc1_ref[...],
                             preferred_element_type=jnp.float32) + bc1_ref[...], 0.0)
    g2 = jnp.maximum(jnp.dot(g1, wc2_ref[...],
                             preferred_element_type=jnp.float32) + bc2_ref[...], 0.0)
    logits = jnp.dot(g2, wc3_ref[...],
                     preferred_element_type=jnp.float32) + bc3_ref[...]
    m = jnp.max(logits, axis=1, keepdims=True)
    e = jnp.exp(logits - m)
    lse = jnp.log(jnp.sum(e, axis=1, keepdims=True))
    out_ref[...] = logits - m - lse


_tc_head = pl.pallas_call(
    _tc_head_body,
    out_shape=(jax.ShapeDtypeStruct((N, H), jnp.float32),
               jax.ShapeDtypeStruct((2, G), jnp.int32)),
)

_tc_mid = pl.pallas_call(
    _tc_mid_body,
    out_shape=jax.ShapeDtypeStruct((N, H), jnp.float32),
)

_tc_tail = pl.pallas_call(
    _tc_tail_body,
    out_shape=jax.ShapeDtypeStruct((G, 2), jnp.float32),
    in_specs=[pl.BlockSpec(memory_space=pltpu.VMEM)] * 7
    + [pl.BlockSpec(memory_space=pltpu.SMEM)]
    + [pl.BlockSpec(memory_space=pltpu.VMEM)] * 6,
    scratch_shapes=[pltpu.VMEM((G, H), jnp.float32),
                    pltpu.VMEM((N, H), jnp.float32),
                    pltpu.VMEM((NB, H), jnp.float32)],
)


def kernel(x, edge_index, batch, W1, b1, W2, b2, W3, b3,
           Wc1, bc1, Wc2, bc2, Wc3, bc3):
    pidx = jnp.arange(EPAD, dtype=jnp.int32)
    pad = jnp.concatenate(
        [(pidx * 8 % N).reshape(1, EPAD),
         (N + pidx % 16).reshape(1, EPAD)], axis=0)
    ei3d = jnp.concatenate([edge_index, pad], axis=1).reshape(2, NW, NCHUNK, CH)
    brow = batch.reshape(1, N)
    bcol = batch.reshape(N, 1)
    batch16 = batch.reshape(NB, 16)

    sc_degree, sc_aggregate = _sc_kernels()
    degp = sc_degree(ei3d)
    hs1, se = _tc_head(x, W1, degp, bcol)
    p1 = sc_aggregate(hs1, ei3d)
    hs2 = _tc_mid(p1, hs1, degp, W2, b1.reshape(1, H))
    p2 = sc_aggregate(hs2, ei3d)
    hs3 = _tc_mid(p2, hs2, degp, W3, b2.reshape(1, H))
    p3 = sc_aggregate(hs3, ei3d)
    return _tc_tail(p3, hs3, degp, b3.reshape(1, H), brow, bcol, batch16, se,
                    Wc1, bc1.reshape(1, H), Wc2, bc2.reshape(1, H // 2),
                    Wc3, bc3.reshape(1, 2))
